# baseline (speedup 1.0000x reference)
import jax
import jax.numpy as jnp
from jax import lax
from jax.experimental import pallas as pl
from jax.experimental.pallas import tpu as pltpu
from jax.experimental.pallas import tpu_sc as plsc

B, L, V = 1024, 50, 64
ROWS = B * L
P = 0.1
C_SUB = 63.0 / 62.0
C_FILL = 1.0 / 62.0
PA = 1.0 - P - P / 62.0
PB = P / 62.0

NC, NS = 2, 16
NW = NC * NS
RPW = ROWS // NW
BLK = 160
NBLK = RPW // BLK
NG = BLK // 16


def _bit31(j):
    c = 1 << (j % 32)
    if c >= 2**31:
        c -= 2**32
    return jnp.int32(c)


def _body(m_hbm, p_hbm, k_hbm, nm_hbm, np_hbm, mc_hbm, pc_hbm,
          m_v, p_v, k_v, nm_v, np_v, t_v, sem_m, sem_p):
    cid = lax.axis_index("c")
    sid = lax.axis_index("s")
    wid = sid * NC + cid

    def block(b, carry):
        base = wid * RPW + b * BLK
        pltpu.sync_copy(m_hbm.at[pl.ds(base * V, BLK * V)], m_v)
        pltpu.sync_copy(p_hbm.at[pl.ds(base * V, BLK * V)], p_v)
        pltpu.sync_copy(k_hbm.at[pl.ds(base * 2, BLK * 2)], k_v)
        cp_m = pltpu.make_async_copy(m_v, mc_hbm.at[pl.ds(base * V, BLK * V)],
                                     sem_m)
        cp_p = pltpu.make_async_copy(p_v, pc_hbm.at[pl.ds(base * V, BLK * V)],
                                     sem_p)
        cp_m.start()
        cp_p.start()

        def group(g, gcarry):
            rbase = g * 16 + lax.iota(jnp.int32, 16)
            rowsV = rbase * V
            rows2 = rbase * 2
            w0 = plsc.load_gather(k_v, [rows2])
            w1 = plsc.load_gather(k_v, [rows2 + 1])
            fill = jnp.zeros((16,), jnp.float32)
            prev_w = jnp.zeros((16,), jnp.float32)
            for j in range(64):
                mj = plsc.load_gather(m_v, [rowsV + j])
                if j == 0:
                    t_v[pl.ds(0, 16)] = mj
                else:
                    t_v[pl.ds(j * 16, 16)] = mj - C_SUB * prev_w
                if j < 63:
                    w = w0 if j < 32 else w1
                    bit = w & _bit31(j)
                    prev_w = jnp.where(bit != jnp.int32(0), mj, 0.0)
                    fill = fill + prev_w
            fillv = fill * C_FILL
            for j in range(64):
                o = t_v[pl.ds(j * 16, 16)]
                if j > 0:
                    o = o + fillv
                plsc.store_scatter(nm_v, [rowsV + j], o)
            p0 = plsc.load_gather(p_v, [rowsV])
            tb = PB * (1.0 - p0)
            plsc.store_scatter(np_v, [rowsV], p0)
            for j in range(1, 64):
                pj = plsc.load_gather(p_v, [rowsV + j])
                plsc.store_scatter(np_v, [rowsV + j], PA * pj + tb)
            return gcarry

        lax.fori_loop(0, NG, group, 0)
        cp_m.wait()
        cp_p.wait()
        pltpu.sync_copy(nm_v, nm_hbm.at[pl.ds(base * V, BLK * V)])
        pltpu.sync_copy(np_v, np_hbm.at[pl.ds(base * V, BLK * V)])
        return carry

    lax.fori_loop(0, NBLK, block, 0)


@jax.jit
def _sc_call(m1, p1, packed1):
    mesh = plsc.VectorSubcoreMesh(core_axis_name="c", subcore_axis_name="s")
    f32 = jnp.float32
    kern = pl.kernel(
        _body,
        out_type=[
            jax.ShapeDtypeStruct((ROWS * V,), f32),
            jax.ShapeDtypeStruct((ROWS * V,), f32),
            jax.ShapeDtypeStruct((ROWS * V,), f32),
            jax.ShapeDtypeStruct((ROWS * V,), f32),
        ],
        mesh=mesh,
        scratch_types=[
            pltpu.VMEM((BLK * V,), f32),
            pltpu.VMEM((BLK * V,), f32),
            pltpu.VMEM((BLK * 2,), jnp.int32),
            pltpu.VMEM((BLK * V,), f32),
            pltpu.VMEM((BLK * V,), f32),
            pltpu.VMEM((V * 16,), f32),
            pltpu.SemaphoreType.DMA,
            pltpu.SemaphoreType.DMA,
        ],
        compiler_params=pltpu.CompilerParams(needs_layout_passes=False),
    )
    return kern(m1, p1, packed1)


def kernel(messages, probs, target_mask):
    m1 = messages.reshape(ROWS * V)
    p1 = probs.reshape(ROWS * V)
    bits = target_mask.astype(jnp.uint32)
    lo = jnp.sum(bits[:, :32] << jnp.arange(32, dtype=jnp.uint32)[None, :],
                 axis=-1, dtype=jnp.uint32)
    hi = jnp.sum(bits[:, 32:] << jnp.arange(31, dtype=jnp.uint32)[None, :],
                 axis=-1, dtype=jnp.uint32)
    packed1 = lax.bitcast_convert_type(
        jnp.stack([lo, hi], axis=-1), jnp.int32).reshape(ROWS * 2)
    nm, npp, mc, pc = _sc_call(m1, p1, packed1)
    return (nm.reshape(B, L, V), npp.reshape(B, L, V),
            mc.reshape(B, L, V), pc.reshape(B, L, V))


# speedup vs baseline: 2.3462x; 2.3462x over previous
import jax
import jax.numpy as jnp
from jax import lax
from jax.experimental import pallas as pl
from jax.experimental.pallas import tpu as pltpu
from jax.experimental.pallas import tpu_sc as plsc

B, L, V = 1024, 50, 64
P = 0.1
C_SUB = 63.0 / 62.0
C_FILL = 1.0 / 62.0
PA = 1.0 - P - P / 62.0
PB = P / 62.0

NC, NS = 2, 16
NB = 8
NITER = 9


def _bitc(v):
    c = 1 << (v % 32)
    if c >= 2**31:
        c -= 2**32
    return jnp.int32(c)


def _body(m_hbm, p_hbm, k_hbm, nm_hbm, np_hbm, mc_hbm, pc_hbm,
          m0, m1, p0_, p1_, k0, k1, nm0, nm1, np0, np1,
          ixi0, ixi1, ixo0, ixo1,
          sim0, sim1, sip0, sip1, sik0, sik1,
          scm0, scm1, scp0, scp1, som0, som1, sop0, sop1):
    m_scr = [m0, m1]
    p_scr = [p0_, p1_]
    k_scr = [k0, k1]
    nm_scr = [nm0, nm1]
    np_scr = [np0, np1]
    idx_in = [ixi0, ixi1]
    idx_out = [ixo0, ixo1]
    sin_m = [sim0, sim1]
    sin_p = [sip0, sip1]
    sin_k = [sik0, sik1]
    scp_m = [scm0, scm1]
    scp_p = [scp0, scp1]
    sout_m = [som0, som1]
    sout_p = [sop0, sop1]

    cid = lax.axis_index("c")
    sid = lax.axis_index("s")
    wid = sid * NC + cid
    bh = lax.rem(wid, NB)
    q = wid // NB
    l0 = q * 13 - jnp.maximum(q - 2, 0)
    n = jnp.where(q < 2, 13, 12)

    def set_idx(ref, c):
        base = (l0 + c) * 512 + bh
        for g in range(V // 16):
            ref[pl.ds(g * 16, 16)] = (
                base + 8 * (lax.iota(jnp.int32, 16) + g * 16))

    def in_copies(bi, c):
        return (
            pltpu.make_async_copy(m_hbm.at[idx_in[bi]], m_scr[bi],
                                  sin_m[bi]),
            pltpu.make_async_copy(p_hbm.at[idx_in[bi]], p_scr[bi],
                                  sin_p[bi]),
            pltpu.make_async_copy(
                k_hbm.at[pl.ds(((l0 + c) * NB + bh) * 256, 256)],
                k_scr[bi], sin_k[bi]),
        )

    def out_copy(bi, which):
        scr, hbm, sem = {
            "cm": (m_scr[bi], mc_hbm, scp_m[bi]),
            "cp": (p_scr[bi], pc_hbm, scp_p[bi]),
            "nm": (nm_scr[bi], nm_hbm, sout_m[bi]),
            "np": (np_scr[bi], np_hbm, sout_p[bi]),
        }[which]
        return pltpu.make_async_copy(scr, hbm.at[idx_out[bi]], sem)

    def compute(bi):
        mv_s = m_scr[bi]
        pv_s = p_scr[bi]
        kv_s = k_scr[bi]
        nmv_s = nm_scr[bi]
        npv_s = np_scr[bi]

        def do_s(s, carry):
            lanes = pl.ds(s * 16, 16)
            w0 = kv_s[pl.ds(s * 16, 16)]
            w1 = kv_s[pl.ds(128 + s * 16, 16)]
            facc = [jnp.zeros((16,), jnp.float32) for _ in range(4)]
            prevw = jnp.zeros((16,), jnp.float32)
            for v in range(V):
                mv = mv_s[v, lanes]
                if v == 0:
                    nmv_s[v, lanes] = mv
                else:
                    nmv_s[v, lanes] = mv - C_SUB * prevw
                if v < V - 1:
                    w = w0 if v < 32 else w1
                    bit = w & _bitc(v)
                    prevw = jnp.where(bit != jnp.int32(0), mv, 0.0)
                    facc[v % 4] = facc[v % 4] + prevw
            fillv = ((facc[0] + facc[1]) + (facc[2] + facc[3])) * C_FILL
            for v in range(1, V):
                plsc.addupdate(nmv_s.at[v, lanes], fillv)
            p0v = pv_s[0, lanes]
            tb = PB * (1.0 - p0v)
            npv_s[0, lanes] = p0v
            for v in range(1, V):
                npv_s[v, lanes] = PA * pv_s[v, lanes] + tb
            return carry

        lax.fori_loop(0, 128 // 16, do_s, 0)

    for bi in range(2):
        set_idx(idx_in[bi], bi)
        for cp in in_copies(bi, bi):
            cp.start()

    def outer(c2, carry):
        for bi in range(2):
            c = c2 * 2 + bi

            @pl.when(c < n)
            def _wait_in():
                for cp in in_copies(bi, c):
                    cp.wait()

            @pl.when(jnp.logical_and(c2 > 0, c - 2 < n))
            def _wait_prev_out():
                out_copy(bi, "nm").wait()
                out_copy(bi, "np").wait()

            @pl.when(c < n)
            def _work():
                set_idx(idx_out[bi], c)
                out_copy(bi, "cm").start()
                out_copy(bi, "cp").start()
                compute(bi)
                out_copy(bi, "nm").start()
                out_copy(bi, "np").start()
                out_copy(bi, "cm").wait()
                out_copy(bi, "cp").wait()

            @pl.when(c + 2 < n)
            def _start_next_in():
                set_idx(idx_in[bi], c + 2)
                for cp in in_copies(bi, c + 2):
                    cp.start()
        return carry

    lax.fori_loop(0, NITER, outer, 0)


@jax.jit
def _sc_call(mt, pt, pk):
    mesh = plsc.VectorSubcoreMesh(core_axis_name="c", subcore_axis_name="s")
    f32 = jnp.float32
    slab = pltpu.VMEM((V, 128), f32)
    kslab = pltpu.VMEM((256,), jnp.int32)
    ixv = pltpu.VMEM((V,), jnp.int32)
    dma = pltpu.SemaphoreType.DMA
    kern = pl.kernel(
        _body,
        out_type=[
            jax.ShapeDtypeStruct((L * V * NB, 128), f32),
            jax.ShapeDtypeStruct((L * V * NB, 128), f32),
            jax.ShapeDtypeStruct((L * V * NB, 128), f32),
            jax.ShapeDtypeStruct((L * V * NB, 128), f32),
        ],
        mesh=mesh,
        scratch_types=[slab, slab, slab, slab, kslab, kslab,
                       slab, slab, slab, slab, ixv, ixv, ixv, ixv]
                      + [dma] * 14,
        compiler_params=pltpu.CompilerParams(needs_layout_passes=False),
    )
    return kern(mt, pt, pk)


def kernel(messages, probs, target_mask):
    mt = messages.transpose(1, 2, 0).reshape(L * V * NB, 128)
    pt = probs.transpose(1, 2, 0).reshape(L * V * NB, 128)
    tm3 = target_mask.reshape(B, L, V - 1)
    bits = tm3.astype(jnp.uint32)
    lo = jnp.sum(bits[:, :, :32] << jnp.arange(32, dtype=jnp.uint32),
                 axis=-1, dtype=jnp.uint32)
    hi = jnp.sum(bits[:, :, 32:] << jnp.arange(31, dtype=jnp.uint32),
                 axis=-1, dtype=jnp.uint32)
    pk = jnp.stack([lo.T.reshape(L, NB, 128),
                    hi.T.reshape(L, NB, 128)], axis=2)
    pk = lax.bitcast_convert_type(pk, jnp.int32).reshape(L * NB * 256)
    nm, npp, mc, pc = _sc_call(mt, pt, pk)

    def back(x):
        return x.reshape(L, V, B).transpose(2, 0, 1)

    return (back(nm), back(npp), back(mc), back(pc))


# speedup vs baseline: 3.7580x; 1.6017x over previous
import jax
import jax.numpy as jnp
from jax import lax
from jax.experimental import pallas as pl
from jax.experimental.pallas import tpu as pltpu
from jax.experimental.pallas import tpu_sc as plsc

B, L, V = 1024, 50, 64
P = 0.1
C_SUB = 63.0 / 62.0
C_FILL = 1.0 / 62.0
PA = 1.0 - P - P / 62.0
PB = P / 62.0

NC, NS = 2, 16
NB = 8
NITER = 9


def _bitc(v):
    c = 1 << (v % 32)
    if c >= 2**31:
        c -= 2**32
    return jnp.int32(c)


def _body(m_hbm, p_hbm, k_hbm, nm_hbm, np_hbm, mc_hbm, pc_hbm,
          m0, m1, p0_, p1_, k0, k1, nm0, nm1, np0, np1,
          ixi0, ixi1, ixo0, ixo1,
          sim0, sim1, sip0, sip1, sik0, sik1,
          scm0, scm1, scp0, scp1, som0, som1, sop0, sop1):
    m_scr = [m0, m1]
    p_scr = [p0_, p1_]
    k_scr = [k0, k1]
    nm_scr = [nm0, nm1]
    np_scr = [np0, np1]
    idx_in = [ixi0, ixi1]
    idx_out = [ixo0, ixo1]
    sin_m = [sim0, sim1]
    sin_p = [sip0, sip1]
    sin_k = [sik0, sik1]
    scp_m = [scm0, scm1]
    scp_p = [scp0, scp1]
    sout_m = [som0, som1]
    sout_p = [sop0, sop1]

    cid = lax.axis_index("c")
    sid = lax.axis_index("s")
    wid = sid * NC + cid
    bh = lax.rem(wid, NB)
    q = wid // NB
    l0 = q * 13 - jnp.maximum(q - 2, 0)
    n = jnp.where(q < 2, 13, 12)

    def set_idx(ref, c):
        base = (l0 + c) * 512 + bh * 8
        it = lax.iota(jnp.int32, 16)
        voff = ((it >> 3) << 6) + (it & 7)
        for g in range(V // 16):
            ref[pl.ds(g * 16, 16)] = base + g * 128 + voff

    def in_copies(bi, c):
        return (
            pltpu.make_async_copy(m_hbm.at[idx_in[bi]], m_scr[bi],
                                  sin_m[bi]),
            pltpu.make_async_copy(p_hbm.at[idx_in[bi]], p_scr[bi],
                                  sin_p[bi]),
            pltpu.make_async_copy(
                k_hbm.at[pl.ds(((l0 + c) * NB + bh) * 256, 256)],
                k_scr[bi], sin_k[bi]),
        )

    def out_copy(bi, which):
        scr, hbm, sem = {
            "cm": (m_scr[bi], mc_hbm, scp_m[bi]),
            "cp": (p_scr[bi], pc_hbm, scp_p[bi]),
            "nm": (nm_scr[bi], nm_hbm, sout_m[bi]),
            "np": (np_scr[bi], np_hbm, sout_p[bi]),
        }[which]
        return pltpu.make_async_copy(scr, hbm.at[idx_out[bi]], sem)

    def compute(bi):
        mv_s = m_scr[bi]
        pv_s = p_scr[bi]
        kv_s = k_scr[bi]
        nmv_s = nm_scr[bi]
        npv_s = np_scr[bi]

        def do_s(s, carry):
            lanes = pl.ds(s * 16, 16)
            w0 = kv_s[pl.ds(s * 16, 16)]
            w1 = kv_s[pl.ds(128 + s * 16, 16)]
            facc = [jnp.zeros((16,), jnp.float32) for _ in range(4)]
            prevw = jnp.zeros((16,), jnp.float32)
            for v in range(V):
                mv = mv_s[v, lanes]
                if v == 0:
                    nmv_s[v, lanes] = mv
                else:
                    nmv_s[v, lanes] = mv - C_SUB * prevw
                if v < V - 1:
                    w = w0 if v < 32 else w1
                    bit = w & _bitc(v)
                    prevw = jnp.where(bit != jnp.int32(0), mv, 0.0)
                    facc[v % 4] = facc[v % 4] + prevw
            fillv = ((facc[0] + facc[1]) + (facc[2] + facc[3])) * C_FILL
            for v in range(1, V):
                plsc.addupdate(nmv_s.at[v, lanes], fillv)
            p0v = pv_s[0, lanes]
            tb = PB * (1.0 - p0v)
            npv_s[0, lanes] = p0v
            for v in range(1, V):
                npv_s[v, lanes] = PA * pv_s[v, lanes] + tb
            return carry

        lax.fori_loop(0, 128 // 16, do_s, 0)

    for bi in range(2):
        set_idx(idx_in[bi], bi)
        for cp in in_copies(bi, bi):
            cp.start()

    def outer(c2, carry):
        for bi in range(2):
            c = c2 * 2 + bi

            @pl.when(c < n)
            def _wait_in():
                for cp in in_copies(bi, c):
                    cp.wait()

            @pl.when(jnp.logical_and(c2 > 0, c - 2 < n))
            def _wait_prev_out():
                out_copy(bi, "nm").wait()
                out_copy(bi, "np").wait()

            @pl.when(c < n)
            def _work():
                set_idx(idx_out[bi], c)
                out_copy(bi, "cm").start()
                out_copy(bi, "cp").start()
                compute(bi)
                out_copy(bi, "nm").start()
                out_copy(bi, "np").start()
                out_copy(bi, "cm").wait()
                out_copy(bi, "cp").wait()

            @pl.when(c + 2 < n)
            def _start_next_in():
                set_idx(idx_in[bi], c + 2)
                for cp in in_copies(bi, c + 2):
                    cp.start()
        return carry

    lax.fori_loop(0, NITER, outer, 0)


@jax.jit
def _sc_call(mt, pt, pk):
    mesh = plsc.VectorSubcoreMesh(core_axis_name="c", subcore_axis_name="s")
    f32 = jnp.float32
    slab = pltpu.VMEM((V, 128), f32)
    kslab = pltpu.VMEM((256,), jnp.int32)
    ixv = pltpu.VMEM((V,), jnp.int32)
    dma = pltpu.SemaphoreType.DMA
    kern = pl.kernel(
        _body,
        out_type=[
            jax.ShapeDtypeStruct((L * V * NB, 128), f32),
            jax.ShapeDtypeStruct((L * V * NB, 128), f32),
            jax.ShapeDtypeStruct((L * V * NB, 128), f32),
            jax.ShapeDtypeStruct((L * V * NB, 128), f32),
        ],
        mesh=mesh,
        scratch_types=[slab, slab, slab, slab, kslab, kslab,
                       slab, slab, slab, slab, ixv, ixv, ixv, ixv]
                      + [dma] * 14,
        compiler_params=pltpu.CompilerParams(needs_layout_passes=False),
    )
    return kern(mt, pt, pk)


def _to_phys(x):
    return (x.reshape(NB, 128, L, 8, 8)
            .transpose(2, 3, 0, 4, 1)
            .reshape(L * V * NB, 128))


def _from_phys(x):
    return (x.reshape(L, 8, NB, 8, 128)
            .transpose(2, 4, 0, 1, 3)
            .reshape(B, L, V))


def kernel(messages, probs, target_mask):
    mt = _to_phys(messages)
    pt = _to_phys(probs)
    tm3 = target_mask.reshape(B, L, V - 1)
    bits = tm3.astype(jnp.uint32)
    lo = jnp.sum(bits[:, :, :32] << jnp.arange(32, dtype=jnp.uint32),
                 axis=-1, dtype=jnp.uint32)
    hi = jnp.sum(bits[:, :, 32:] << jnp.arange(31, dtype=jnp.uint32),
                 axis=-1, dtype=jnp.uint32)
    pk = jnp.stack([lo.T.reshape(L, NB, 128),
                    hi.T.reshape(L, NB, 128)], axis=2)
    pk = lax.bitcast_convert_type(pk, jnp.int32).reshape(L * NB * 256)
    nm, npp, mc, pc = _sc_call(mt, pt, pk)
    return (_from_phys(nm), _from_phys(npp), _from_phys(mc), _from_phys(pc))


# speedup vs baseline: 6.9848x; 1.8587x over previous
import jax
import jax.numpy as jnp
from jax import lax
from jax.experimental import pallas as pl
from jax.experimental.pallas import tpu as pltpu
from jax.experimental.pallas import tpu_sc as plsc

B, L, V = 1024, 50, 64
P = 0.1
C_SUB = 63.0 / 62.0
C_FILL = 1.0 / 62.0
PA = 1.0 - P - P / 62.0
PB = P / 62.0

NC, NS = 2, 16
NB = 8
NITER = 9


def _bitc(v):
    c = 1 << (v % 32)
    if c >= 2**31:
        c -= 2**32
    return jnp.int32(c)


def _body(m_hbm, p_hbm, k_hbm, nm_hbm, np_hbm, mc_hbm, pc_hbm,
          m0, m1, p0_, p1_, k0, k1, nm0, nm1, np0, np1,
          ixi0, ixi1, ixo0, ixo1,
          sim0, sim1, sip0, sip1, sik0, sik1,
          scm0, scm1, scp0, scp1, som0, som1, sop0, sop1):
    m_scr = [m0, m1]
    p_scr = [p0_, p1_]
    k_scr = [k0, k1]
    nm_scr = [nm0, nm1]
    np_scr = [np0, np1]
    idx_in = [ixi0, ixi1]
    idx_out = [ixo0, ixo1]
    sin_m = [sim0, sim1]
    sin_p = [sip0, sip1]
    sin_k = [sik0, sik1]
    scp_m = [scm0, scm1]
    scp_p = [scp0, scp1]
    sout_m = [som0, som1]
    sout_p = [sop0, sop1]

    cid = lax.axis_index("c")
    sid = lax.axis_index("s")
    wid = sid * NC + cid
    bh = lax.rem(wid, NB)
    q = wid // NB
    l0 = q * 13 - jnp.maximum(q - 2, 0)
    n = jnp.where(q < 2, 13, 12)

    def set_idx(ref, c):
        base = (l0 + c) * 512 + bh * 8
        it = lax.iota(jnp.int32, 16)
        voff = ((it >> 3) << 6) + (it & 7)
        for g in range(V // 16):
            ref[pl.ds(g * 16, 16)] = base + g * 128 + voff

    def in_copies(bi, c):
        return (
            pltpu.make_async_copy(m_hbm.at[idx_in[bi]], m_scr[bi],
                                  sin_m[bi]),
            pltpu.make_async_copy(p_hbm.at[idx_in[bi]], p_scr[bi],
                                  sin_p[bi]),
            pltpu.make_async_copy(
                k_hbm.at[pl.ds(((l0 + c) * NB + bh) * 256, 256)],
                k_scr[bi], sin_k[bi]),
        )

    def out_copy(bi, which):
        scr, hbm, sem = {
            "cm": (m_scr[bi], mc_hbm, scp_m[bi]),
            "cp": (p_scr[bi], pc_hbm, scp_p[bi]),
            "nm": (nm_scr[bi], nm_hbm, sout_m[bi]),
            "np": (np_scr[bi], np_hbm, sout_p[bi]),
        }[which]
        return pltpu.make_async_copy(scr, hbm.at[idx_out[bi]], sem)

    def compute(bi):
        mv_s = m_scr[bi]
        pv_s = p_scr[bi]
        kv_s = k_scr[bi]
        nmv_s = nm_scr[bi]
        npv_s = np_scr[bi]

        def do_s(s, carry):
            lanes = pl.ds(s * 16, 16)
            w0 = kv_s[pl.ds(s * 16, 16)]
            w1 = kv_s[pl.ds(128 + s * 16, 16)]
            facc = [jnp.zeros((16,), jnp.float32) for _ in range(4)]
            prevw = jnp.zeros((16,), jnp.float32)
            for v in range(V):
                mv = mv_s[v, lanes]
                if v == 0:
                    nmv_s[v, lanes] = mv
                else:
                    nmv_s[v, lanes] = mv - C_SUB * prevw
                if v < V - 1:
                    w = w0 if v < 32 else w1
                    bit = w & _bitc(v)
                    prevw = jnp.where(bit != jnp.int32(0), mv, 0.0)
                    facc[v % 4] = facc[v % 4] + prevw
            fillv = ((facc[0] + facc[1]) + (facc[2] + facc[3])) * C_FILL
            for v in range(1, V):
                plsc.addupdate(nmv_s.at[v, lanes], fillv)
            p0v = pv_s[0, lanes]
            tb = PB * (1.0 - p0v)
            npv_s[0, lanes] = p0v
            for v in range(1, V):
                npv_s[v, lanes] = PA * pv_s[v, lanes] + tb
            return carry

        lax.fori_loop(0, 128 // 16, do_s, 0)

    for bi in range(2):
        set_idx(idx_in[bi], bi)
        for cp in in_copies(bi, bi):
            cp.start()

    def outer(c2, carry):
        for bi in range(2):
            c = c2 * 2 + bi

            @pl.when(c < n)
            def _wait_in():
                for cp in in_copies(bi, c):
                    cp.wait()

            @pl.when(jnp.logical_and(c2 > 0, c - 2 < n))
            def _wait_prev_out():
                out_copy(bi, "nm").wait()
                out_copy(bi, "np").wait()

            @pl.when(c < n)
            def _work():
                set_idx(idx_out[bi], c)
                out_copy(bi, "cm").start()
                out_copy(bi, "cp").start()
                compute(bi)
                out_copy(bi, "nm").start()
                out_copy(bi, "np").start()
                out_copy(bi, "cm").wait()
                out_copy(bi, "cp").wait()

            @pl.when(c + 2 < n)
            def _start_next_in():
                set_idx(idx_in[bi], c + 2)
                for cp in in_copies(bi, c + 2):
                    cp.start()
        return carry

    lax.fori_loop(0, NITER, outer, 0)


@jax.jit
def _sc_call(mt, pt, pk):
    mesh = plsc.VectorSubcoreMesh(core_axis_name="c", subcore_axis_name="s")
    f32 = jnp.float32
    slab = pltpu.VMEM((V, 128), f32)
    kslab = pltpu.VMEM((256,), jnp.int32)
    ixv = pltpu.VMEM((V,), jnp.int32)
    dma = pltpu.SemaphoreType.DMA
    kern = pl.kernel(
        _body,
        out_type=[
            jax.ShapeDtypeStruct((L * V * NB, 128), f32),
            jax.ShapeDtypeStruct((L * V * NB, 128), f32),
            jax.ShapeDtypeStruct((L * V * NB, 128), f32),
            jax.ShapeDtypeStruct((L * V * NB, 128), f32),
        ],
        mesh=mesh,
        scratch_types=[slab, slab, slab, slab, kslab, kslab,
                       slab, slab, slab, slab, ixv, ixv, ixv, ixv]
                      + [dma] * 14,
        compiler_params=pltpu.CompilerParams(needs_layout_passes=False),
    )
    return kern(mt, pt, pk)


def _to_phys(x):
    return (x.reshape(NB, 128, L, 8, 8)
            .transpose(2, 3, 0, 4, 1)
            .reshape(L * V * NB, 128))


def _from_phys(x):
    return (x.reshape(L, 8, NB, 8, 128)
            .transpose(2, 4, 0, 1, 3)
            .reshape(B, L, V))


def kernel(messages, probs, target_mask):
    mt = _to_phys(messages)
    pt = _to_phys(probs)
    bits = target_mask.T.astype(jnp.uint32)
    lo = jnp.sum(bits[:32] << jnp.arange(32, dtype=jnp.uint32)[:, None],
                 axis=0, dtype=jnp.uint32)
    hi = jnp.sum(bits[32:] << jnp.arange(31, dtype=jnp.uint32)[:, None],
                 axis=0, dtype=jnp.uint32)
    loT = lo.reshape(B, L).T
    hiT = hi.reshape(B, L).T
    pk = jnp.stack([loT.reshape(L, NB, 128),
                    hiT.reshape(L, NB, 128)], axis=2)
    pk = lax.bitcast_convert_type(pk, jnp.int32).reshape(L * NB * 256)
    nm, npp, mc, pc = _sc_call(mt, pt, pk)
    return (_from_phys(nm), _from_phys(npp), _from_phys(mc), _from_phys(pc))


# speedup vs baseline: 8.3170x; 1.1907x over previous
import jax
import jax.numpy as jnp
from jax import lax
from jax.experimental import pallas as pl
from jax.experimental.pallas import tpu as pltpu
from jax.experimental.pallas import tpu_sc as plsc

B, L, V = 1024, 50, 64
P = 0.1
C_SUB = 63.0 / 62.0
C_FILL = 1.0 / 62.0
PA = 1.0 - P - P / 62.0
PB = P / 62.0

NC, NS = 2, 16
NB = 8
NITER = 9


def _bitc(v):
    c = 1 << (v % 32)
    if c >= 2**31:
        c -= 2**32
    return jnp.int32(c)


def _body(m_hbm, p_hbm, k_hbm, nm_hbm, np_hbm, mc_hbm, pc_hbm,
          m0, m1, p0_, p1_, k0, k1, nm0, nm1, np0, np1,
          ixi0, ixi1, ixo0, ixo1,
          sim0, sim1, sip0, sip1, sik0, sik1,
          scm0, scm1, scp0, scp1, som0, som1, sop0, sop1):
    m_scr = [m0, m1]
    p_scr = [p0_, p1_]
    k_scr = [k0, k1]
    nm_scr = [nm0, nm1]
    np_scr = [np0, np1]
    idx_in = [ixi0, ixi1]
    idx_out = [ixo0, ixo1]
    sin_m = [sim0, sim1]
    sin_p = [sip0, sip1]
    sin_k = [sik0, sik1]
    scp_m = [scm0, scm1]
    scp_p = [scp0, scp1]
    sout_m = [som0, som1]
    sout_p = [sop0, sop1]

    cid = lax.axis_index("c")
    sid = lax.axis_index("s")
    wid = sid * NC + cid
    bh = lax.rem(wid, NB)
    q = wid // NB
    l0 = q * 13 - jnp.maximum(q - 2, 0)
    n = jnp.where(q < 2, 13, 12)

    def set_idx(ref, c):
        base = (l0 + c) * 512 + bh * 8
        it = lax.iota(jnp.int32, 16)
        voff = ((it >> 3) << 6) + (it & 7)
        for g in range(V // 16):
            ref[pl.ds(g * 16, 16)] = base + g * 128 + voff

    def in_copies(bi, c):
        return (
            pltpu.make_async_copy(m_hbm.at[idx_in[bi]], m_scr[bi],
                                  sin_m[bi]),
            pltpu.make_async_copy(p_hbm.at[idx_in[bi]], p_scr[bi],
                                  sin_p[bi]),
            pltpu.make_async_copy(
                k_hbm.at[pl.ds(((l0 + c) * NB + bh) * 256, 256)],
                k_scr[bi], sin_k[bi]),
        )

    def out_copy(bi, which):
        scr, hbm, sem = {
            "cm": (m_scr[bi], mc_hbm, scp_m[bi]),
            "cp": (p_scr[bi], pc_hbm, scp_p[bi]),
            "nm": (nm_scr[bi], nm_hbm, sout_m[bi]),
            "np": (np_scr[bi], np_hbm, sout_p[bi]),
        }[which]
        return pltpu.make_async_copy(scr, hbm.at[idx_out[bi]], sem)

    def compute(bi):
        mv_s = m_scr[bi]
        pv_s = p_scr[bi]
        kv_s = k_scr[bi]
        nmv_s = nm_scr[bi]
        npv_s = np_scr[bi]

        def do_s(s2, carry):
            lanes = [pl.ds(s2 * 32, 16), pl.ds(s2 * 32 + 16, 16)]
            w0 = [kv_s[pl.ds(s2 * 32 + g * 16, 16)] for g in range(2)]
            w1 = [kv_s[pl.ds(128 + s2 * 32 + g * 16, 16)] for g in range(2)]
            facc = [[jnp.zeros((16,), jnp.float32) for _ in range(4)]
                    for _ in range(2)]
            prevw = [jnp.zeros((16,), jnp.float32) for _ in range(2)]
            tb = []
            for g in range(2):
                p0v = pv_s[0, lanes[g]]
                tb.append(PB * (1.0 - p0v))
                npv_s[0, lanes[g]] = p0v
            for v in range(V):
                for g in range(2):
                    mv = mv_s[v, lanes[g]]
                    if v == 0:
                        nmv_s[v, lanes[g]] = mv
                    else:
                        nmv_s[v, lanes[g]] = mv - C_SUB * prevw[g]
                        npv_s[v, lanes[g]] = PA * pv_s[v, lanes[g]] + tb[g]
                    if v < V - 1:
                        w = w0[g] if v < 32 else w1[g]
                        bit = w & _bitc(v)
                        prevw[g] = jnp.where(bit != jnp.int32(0), mv, 0.0)
                        facc[g][v % 4] = facc[g][v % 4] + prevw[g]
            for g in range(2):
                fillv = ((facc[g][0] + facc[g][1])
                         + (facc[g][2] + facc[g][3])) * C_FILL
                for v in range(1, V):
                    plsc.addupdate(nmv_s.at[v, lanes[g]], fillv)
            return carry

        lax.fori_loop(0, 128 // 32, do_s, 0)

    for bi in range(2):
        set_idx(idx_in[bi], bi)
        for cp in in_copies(bi, bi):
            cp.start()

    def outer(c2, carry):
        for bi in range(2):
            c = c2 * 2 + bi

            @pl.when(c < n)
            def _wait_in():
                for cp in in_copies(bi, c):
                    cp.wait()

            @pl.when(jnp.logical_and(c2 > 0, c - 2 < n))
            def _wait_prev_out():
                out_copy(bi, "nm").wait()
                out_copy(bi, "np").wait()

            @pl.when(c < n)
            def _work():
                set_idx(idx_out[bi], c)
                out_copy(bi, "cm").start()
                out_copy(bi, "cp").start()
                compute(bi)
                out_copy(bi, "nm").start()
                out_copy(bi, "np").start()
                out_copy(bi, "cm").wait()
                out_copy(bi, "cp").wait()

            @pl.when(c + 2 < n)
            def _start_next_in():
                set_idx(idx_in[bi], c + 2)
                for cp in in_copies(bi, c + 2):
                    cp.start()
        return carry

    lax.fori_loop(0, NITER, outer, 0)


@jax.jit
def _sc_call(mt, pt, pk):
    mesh = plsc.VectorSubcoreMesh(core_axis_name="c", subcore_axis_name="s")
    f32 = jnp.float32
    slab = pltpu.VMEM((V, 128), f32)
    kslab = pltpu.VMEM((256,), jnp.int32)
    ixv = pltpu.VMEM((V,), jnp.int32)
    dma = pltpu.SemaphoreType.DMA
    kern = pl.kernel(
        _body,
        out_type=[
            jax.ShapeDtypeStruct((L * V * NB, 128), f32),
            jax.ShapeDtypeStruct((L * V * NB, 128), f32),
            jax.ShapeDtypeStruct((L * V * NB, 128), f32),
            jax.ShapeDtypeStruct((L * V * NB, 128), f32),
        ],
        mesh=mesh,
        scratch_types=[slab, slab, slab, slab, kslab, kslab,
                       slab, slab, slab, slab, ixv, ixv, ixv, ixv]
                      + [dma] * 14,
        compiler_params=pltpu.CompilerParams(needs_layout_passes=False),
    )
    return kern(mt, pt, pk)


def _to_phys(x):
    return (x.reshape(NB, 128, L, 8, 8)
            .transpose(2, 3, 0, 4, 1)
            .reshape(L * V * NB, 128))


def _from_phys(x):
    return (x.reshape(L, 8, NB, 8, 128)
            .transpose(2, 4, 0, 1, 3)
            .reshape(B, L, V))


def kernel(messages, probs, target_mask):
    mt = _to_phys(messages)
    pt = _to_phys(probs)
    bits = target_mask.T.astype(jnp.uint32)
    lo = jnp.sum(bits[:32] << jnp.arange(32, dtype=jnp.uint32)[:, None],
                 axis=0, dtype=jnp.uint32)
    hi = jnp.sum(bits[32:] << jnp.arange(31, dtype=jnp.uint32)[:, None],
                 axis=0, dtype=jnp.uint32)
    loT = lo.reshape(B, L).T
    hiT = hi.reshape(B, L).T
    pk = jnp.stack([loT.reshape(L, NB, 128),
                    hiT.reshape(L, NB, 128)], axis=2)
    pk = lax.bitcast_convert_type(pk, jnp.int32).reshape(L * NB * 256)
    nm, npp, mc, pc = _sc_call(mt, pt, pk)
    return (_from_phys(nm), _from_phys(npp), _from_phys(mc), _from_phys(pc))
